# baseline (speedup 1.0000x reference)
import functools

import jax
import jax.numpy as jnp
from jax import lax
from jax.experimental import pallas as pl
from jax.experimental.pallas import tpu as pltpu
from jax.experimental.pallas import tpu_sc as plsc

BATCH = 16384
OUT_DIM = 64

_NUM_CORES = 2
_NUM_SUBCORES = 16
_NUM_WORKERS = _NUM_CORES * _NUM_SUBCORES
_B_PER_W = BATCH // _NUM_WORKERS
_CHUNK = 64
_N_CHUNKS = _B_PER_W // _CHUNK
_LANES = 16
_SLAB = 8

_mesh = plsc.VectorSubcoreMesh(core_axis_name="c", subcore_axis_name="s")


@functools.partial(
    pl.kernel,
    out_type=jax.ShapeDtypeStruct((BATCH, OUT_DIM), jnp.float32),
    mesh=_mesh,
    compiler_params=pltpu.CompilerParams(needs_layout_passes=False),
    scratch_types=[
        pltpu.VMEM((_B_PER_W,), jnp.int32),
        pltpu.VMEM((_CHUNK, _SLAB, OUT_DIM), jnp.float32),
        pltpu.VMEM((_CHUNK, OUT_DIM), jnp.float32),
        pltpu.SemaphoreType.DMA,
    ],
)
def _embed_gather(labels_hbm, table_hbm, out_hbm, lab_v, slab_v,
                  out_v, sem):
    wid = lax.axis_index("s") * _NUM_CORES + lax.axis_index("c")
    base = wid * _B_PER_W
    pltpu.sync_copy(labels_hbm.at[pl.ds(base, _B_PER_W)], lab_v)

    def chunk_body(k, carry):
        copies = []
        for g in range(_CHUNK // _LANES):
            tv = lab_v[pl.ds(k * _CHUNK + g * _LANES, _LANES)] & (-_SLAB)
            for i in range(_LANES):
                copies.append(
                    pltpu.async_copy(
                        table_hbm.at[pl.ds(pl.multiple_of(tv[i], _SLAB), _SLAB)],
                        slab_v.at[g * _LANES + i],
                        sem,
                    )
                )
        for c in copies:
            c.wait()
        for g in range(_CHUNK // _LANES):
            lv = lab_v[pl.ds(k * _CHUNK + g * _LANES, _LANES)]
            r_vec = lv & (_SLAB - 1)
            i_vec = lax.iota(jnp.int32, _LANES) + g * _LANES
            for c in range(OUT_DIM):
                c_vec = jnp.full((_LANES,), c, jnp.int32)
                x = plsc.load_gather(slab_v, [i_vec, r_vec, c_vec])
                plsc.store_scatter(out_v, [i_vec, c_vec], x)
        pltpu.sync_copy(
            out_v, out_hbm.at[pl.ds(base + k * _CHUNK, _CHUNK)]
        )
        return carry

    lax.fori_loop(0, _N_CHUNKS, chunk_body, 0)


def kernel(labels, table, training=0):
    del training
    return _embed_gather(labels.astype(jnp.int32), table)


# speedup vs baseline: 1.0749x; 1.0749x over previous
import functools

import jax
import jax.numpy as jnp
from jax import lax
from jax.experimental import pallas as pl
from jax.experimental.pallas import tpu as pltpu
from jax.experimental.pallas import tpu_sc as plsc

BATCH = 16384
OUT_DIM = 64

_NUM_CORES = 2
_NUM_SUBCORES = 16
_NUM_WORKERS = _NUM_CORES * _NUM_SUBCORES
_B_PER_W = BATCH // _NUM_WORKERS
_CHUNK = 64
_N_CHUNKS = _B_PER_W // _CHUNK
_LANES = 16
_SLAB = 8

_mesh = plsc.VectorSubcoreMesh(core_axis_name="c", subcore_axis_name="s")


@functools.partial(
    pl.kernel,
    out_type=jax.ShapeDtypeStruct((BATCH, OUT_DIM), jnp.float32),
    mesh=_mesh,
    scratch_types=[
        pltpu.VMEM((_B_PER_W,), jnp.int32),
        pltpu.VMEM((_CHUNK, _SLAB, OUT_DIM), jnp.float32),
        pltpu.VMEM((_CHUNK, OUT_DIM), jnp.float32),
        pltpu.SemaphoreType.DMA,
    ],
)
def _embed_gather(labels_hbm, table_hbm, out_hbm, lab_v, slab_v,
                  out_v, sem):
    wid = lax.axis_index("s") * _NUM_CORES + lax.axis_index("c")
    base = wid * _B_PER_W
    pltpu.sync_copy(labels_hbm.at[pl.ds(base, _B_PER_W)], lab_v)

    def chunk_body(k, carry):
        copies = []
        for g in range(_CHUNK // _LANES):
            tv = lab_v[pl.ds(k * _CHUNK + g * _LANES, _LANES)] & (-_SLAB)
            for i in range(_LANES):
                copies.append(
                    pltpu.async_copy(
                        table_hbm.at[pl.ds(pl.multiple_of(tv[i], _SLAB), _SLAB)],
                        slab_v.at[g * _LANES + i],
                        sem,
                    )
                )
        for c in copies:
            c.wait()
        for g in range(_CHUNK // _LANES):
            rv = lab_v[pl.ds(k * _CHUNK + g * _LANES, _LANES)] & (_SLAB - 1)
            for i in range(_LANES):
                j = g * _LANES + i
                r = rv[i]
                for c4 in range(OUT_DIM // _LANES):
                    out_v[j, pl.ds(c4 * _LANES, _LANES)] = (
                        slab_v[j, r, pl.ds(c4 * _LANES, _LANES)]
                    )
        pltpu.sync_copy(
            out_v, out_hbm.at[pl.ds(base + k * _CHUNK, _CHUNK)]
        )
        return carry

    lax.fori_loop(0, _N_CHUNKS, chunk_body, 0)


def kernel(labels, table, training=0):
    del training
    return _embed_gather(labels.astype(jnp.int32), table)


# speedup vs baseline: 1.1652x; 1.0840x over previous
import functools

import jax
import jax.numpy as jnp
from jax import lax
from jax.experimental import pallas as pl
from jax.experimental.pallas import tpu as pltpu
from jax.experimental.pallas import tpu_sc as plsc

BATCH = 16384
OUT_DIM = 64

_NUM_CORES = 2
_NUM_SUBCORES = 16
_NUM_WORKERS = _NUM_CORES * _NUM_SUBCORES
_B_PER_W = BATCH // _NUM_WORKERS
_CHUNK = 128
_N_CHUNKS = _B_PER_W // _CHUNK
_LANES = 16

_mesh = plsc.VectorSubcoreMesh(core_axis_name="c", subcore_axis_name="s")


@functools.partial(
    pl.kernel,
    out_type=jax.ShapeDtypeStruct((BATCH, OUT_DIM), jnp.float32),
    mesh=_mesh,
    scratch_types=[
        pltpu.VMEM((_B_PER_W,), jnp.int32),
        pltpu.VMEM((_CHUNK, OUT_DIM), jnp.float32),
        pltpu.SemaphoreType.DMA,
    ],
)
def _embed_gather(labels_hbm, table_hbm, out_hbm, lab_v, rows_v, sem):
    wid = lax.axis_index("s") * _NUM_CORES + lax.axis_index("c")
    base = wid * _B_PER_W
    pltpu.sync_copy(labels_hbm.at[pl.ds(base, _B_PER_W)], lab_v)

    def chunk_body(k, carry):
        copies = []
        for g in range(_CHUNK // _LANES):
            lv = lab_v[pl.ds(k * _CHUNK + g * _LANES, _LANES)]
            for i in range(_LANES):
                j = g * _LANES + i
                copies.append(
                    pltpu.async_copy(
                        table_hbm.at[pl.ds(lv[i], 1)],
                        rows_v.at[pl.ds(j, 1)],
                        sem,
                    )
                )
        for c in copies:
            c.wait()
        pltpu.sync_copy(
            rows_v, out_hbm.at[pl.ds(base + k * _CHUNK, _CHUNK)]
        )
        return carry

    lax.fori_loop(0, _N_CHUNKS, chunk_body, 0)


def kernel(labels, table, training=0):
    del training
    return _embed_gather(labels.astype(jnp.int32), table)
